# baseline (speedup 1.0000x reference)
import functools

import jax
import jax.numpy as jnp
from jax import lax
from jax.experimental import pallas as pl
from jax.experimental.pallas import tpu as pltpu
from jax.experimental.pallas import tpu_sc as plsc

NUM_ROWS = 16384
NUM_COLS = 26
DIM = 64
V = 1000000

NC = 2
NS = 16
NW = NC * NS

NB = V // 128
NBW = NB // NW
NB_EXTRA = NB - NBW * NW

RB = NUM_ROWS // 128
RBW = RB // NW
BLK_B = NUM_COLS * RBW


def _iota16():
    return lax.iota(jnp.int32, 16)


@functools.partial(
    pl.kernel,
    mesh=plsc.VectorSubcoreMesh(core_axis_name="c", subcore_axis_name="s"),
    out_type=jax.ShapeDtypeStruct((V // 2, 128), jnp.float32),
    scratch_types=[
        pltpu.VMEM((64, 128), jnp.float32),
        pltpu.VMEM((64, 128), jnp.float32),
        pltpu.VMEM((64, 128), jnp.float32),
        pltpu.VMEM((64, 128), jnp.float32),
        pltpu.SemaphoreType.DMA((2,)),
        pltpu.SemaphoreType.DMA((2,)),
    ],
    compiler_params=pltpu.CompilerParams(
        use_tc_tiling_on_sc=True, needs_layout_passes=False, disable_bounds_checks=True
    ),
)
def _relayout_kernel(wt_hbm, tail_hbm, r_hbm, tin0, tin1, tout0, tout1,
                     in_sem, out_sem):
    wid = lax.axis_index("s") * NC + lax.axis_index("c")
    tins = (tin0, tin1)
    touts = (tout0, tout1)
    iota = _iota16()
    row_vecs = [8 * m + iota // 2 for m in range(8)]
    col_base = 64 * (iota % 2)

    def start_in(i, s):
        blk = wid + NW * i
        pltpu.async_copy(
            wt_hbm.at[:, pl.ds(128 * blk, 128)], tins[s], in_sem.at[s]
        )

    def wait_in(s):
        pltpu.make_async_copy(
            wt_hbm.at[:, pl.ds(0, 128)], tins[s], in_sem.at[s]
        ).wait()

    def transpose_block(s):
        tin = tins[s]
        tout = touts[s]
        for row in range(64):
            cols = col_base + row
            for m in range(8):
                val = tin[row, pl.ds(16 * m, 16)]
                plsc.store_scatter(tout, [row_vecs[m], cols], val)

    def start_out(i, s):
        blk = wid + NW * i
        pltpu.async_copy(
            touts[s], r_hbm.at[pl.ds(64 * blk, 64)], out_sem.at[s]
        )

    def wait_out(s):
        pltpu.make_async_copy(
            touts[s], r_hbm.at[pl.ds(0, 64)], out_sem.at[s]
        ).wait()

    start_in(0, 0)
    start_in(1, 1)

    def outer(t, carry):
        for s in range(2):
            i = 2 * t + s
            wait_in(s)

            @pl.when(i >= 2)
            def _():
                wait_out(s)

            transpose_block(s)
            start_out(i, s)

            @pl.when(i + 2 < NBW)
            def _():
                start_in(i + 2, s)
        return carry

    lax.fori_loop(0, NBW // 2, outer, 0)
    wait_out(0)
    wait_out(1)

    @pl.when(wid < NB_EXTRA)
    def _():
        blk = NBW * NW + wid
        pltpu.async_copy(
            wt_hbm.at[:, pl.ds(128 * blk, 128)], tin0, in_sem.at[0]
        )
        wait_in(0)
        transpose_block(0)
        pltpu.async_copy(
            tout0, r_hbm.at[pl.ds(64 * blk, 64)], out_sem.at[0]
        )
        wait_out(0)

    @pl.when(wid == NW - 1)
    def _():
        pltpu.sync_copy(tail_hbm, r_hbm.at[pl.ds((V - 64) // 2, 32)])


@functools.partial(
    pl.kernel,
    mesh=plsc.VectorSubcoreMesh(core_axis_name="c", subcore_axis_name="s"),
    out_type=jax.ShapeDtypeStruct((NUM_COLS, 8, RB, 8, 128), jnp.float32),
    scratch_types=[
        pltpu.VMEM((RBW, NUM_COLS, 128), jnp.int32),
        pltpu.VMEM((128, DIM), jnp.float32),
        pltpu.VMEM((128, DIM), jnp.float32),
        pltpu.VMEM((8, 8, 128), jnp.float32),
        pltpu.VMEM((8, 8, 128), jnp.float32),
        pltpu.SemaphoreType.DMA((2,)),
        pltpu.SemaphoreType.DMA((2,)),
    ],
    compiler_params=pltpu.CompilerParams(
        use_tc_tiling_on_sc=False, needs_layout_passes=False, disable_bounds_checks=True
    ),
)
def _gather_kernel(idx_hbm, table_hbm, out_hbm, idx_v, g0, g1, t0, t1,
                   in_sem, out_sem):
    wid = lax.axis_index("s") * NC + lax.axis_index("c")
    gs = (g0, g1)
    ts = (t0, t1)
    iota = _iota16()
    a_vecs = [(16 * k + iota) // 8 for k in range(4)]
    dr_vec = iota % 8

    pltpu.sync_copy(idx_hbm.at[pl.ds(RBW * wid, RBW)], idx_v)

    def start_in(i, s):
        c = i // RBW
        bl = lax.rem(i, RBW)
        pltpu.async_copy(
            table_hbm.at[idx_v.at[bl, c]], gs[s], in_sem.at[s]
        )

    def wait_in(s):
        pltpu.make_async_copy(
            table_hbm.at[idx_v.at[0, 0]], gs[s], in_sem.at[s]
        ).wait()

    def transpose_block(s):
        g = gs[s]
        t = ts[s]
        for v in range(128):
            vc = jnp.full((16,), v, jnp.int32)
            for k in range(4):
                val = g[v, pl.ds(16 * k, 16)]
                plsc.store_scatter(t, [a_vecs[k], dr_vec, vc], val)

    def start_out(i, s):
        c = i // RBW
        b = RBW * wid + lax.rem(i, RBW)
        pltpu.async_copy(
            ts[s], out_hbm.at[c, :, b], out_sem.at[s]
        )

    def wait_out(s):
        pltpu.make_async_copy(
            ts[s], out_hbm.at[0, :, 0], out_sem.at[s]
        ).wait()

    start_in(0, 0)
    start_in(1, 1)

    def outer(tt, carry):
        for s in range(2):
            i = 2 * tt + s
            wait_in(s)

            @pl.when(i >= 2)
            def _():
                wait_out(s)

            transpose_block(s)
            start_out(i, s)

            @pl.when(i + 2 < BLK_B)
            def _():
                start_in(i + 2, s)
        return carry

    lax.fori_loop(0, BLK_B // 2, outer, 0)
    wait_out(0)
    wait_out(1)


def kernel(x, W):
    wt = jnp.transpose(W)
    tail = W[V - 64:].reshape(32, 128)
    r = _relayout_kernel(wt, tail)
    table = r.reshape(V, DIM)
    idx = x.reshape(RB, 128, NUM_COLS).transpose(0, 2, 1).astype(jnp.int32)
    out5 = _gather_kernel(idx, table)
    return out5.transpose(2, 4, 0, 1, 3).reshape(NUM_ROWS, NUM_COLS, DIM)


# speedup vs baseline: 2.1641x; 2.1641x over previous
import functools

import jax
import jax.numpy as jnp
from jax import lax
from jax.experimental import pallas as pl
from jax.experimental.pallas import tpu as pltpu
from jax.experimental.pallas import tpu_sc as plsc

NUM_ROWS = 16384
NUM_COLS = 26
DIM = 64
V = 1000000

NC = 2
NS = 16
NW = NC * NS

RB = NUM_ROWS // 128
RBW = RB // NW
BLK_B = NUM_COLS * RBW

TCW = 2048
TC_GRID = (V + TCW - 1) // TCW


def _iota16():
    return lax.iota(jnp.int32, 16)


def _tc_relayout_body(wt_ref, out_ref):
    bt = wt_ref[...].T
    sh = jnp.concatenate([bt[1:], bt[:1]], axis=0)
    wide = jnp.concatenate([bt, sh], axis=1)
    out_ref[...] = wide.reshape(TCW // 2, 2, 128)[:, 0, :]


def _tc_relayout(wt):
    return pl.pallas_call(
        _tc_relayout_body,
        grid=(TC_GRID,),
        in_specs=[pl.BlockSpec((DIM, TCW), lambda i: (0, i))],
        out_specs=pl.BlockSpec((TCW // 2, 128), lambda i: (i, 0)),
        out_shape=jax.ShapeDtypeStruct((V // 2, 128), jnp.float32),
    )(wt)


@functools.partial(
    pl.kernel,
    mesh=plsc.VectorSubcoreMesh(core_axis_name="c", subcore_axis_name="s"),
    out_type=jax.ShapeDtypeStruct((NUM_COLS, 8, RB, 8, 128), jnp.float32),
    scratch_types=[
        pltpu.VMEM((RBW, NUM_COLS, 128), jnp.int32),
        pltpu.VMEM((128, DIM), jnp.float32),
        pltpu.VMEM((128, DIM), jnp.float32),
        pltpu.VMEM((DIM, 129), jnp.float32),
        pltpu.VMEM((DIM, 129), jnp.float32),
        pltpu.SemaphoreType.DMA((2,)),
        pltpu.SemaphoreType.DMA((2,)),
    ],
    compiler_params=pltpu.CompilerParams(
        use_tc_tiling_on_sc=False, needs_layout_passes=False,
        disable_bounds_checks=True,
    ),
)
def _gather_kernel(idx_hbm, table_hbm, out_hbm, idx_v, g0, g1, t0, t1,
                   in_sem, out_sem):
    wid = lax.axis_index("s") * NC + lax.axis_index("c")
    gs = (g0, g1)
    ts = (t0, t1)
    iota = _iota16()
    d_vecs = [16 * k + iota for k in range(4)]

    pltpu.sync_copy(idx_hbm.at[pl.ds(RBW * wid, RBW)], idx_v)

    def start_in(i, s):
        c = i // RBW
        bl = lax.rem(i, RBW)
        pltpu.async_copy(
            table_hbm.at[idx_v.at[bl, c]], gs[s], in_sem.at[s]
        )

    def wait_in(s):
        pltpu.make_async_copy(
            table_hbm.at[idx_v.at[0, 0]], gs[s], in_sem.at[s]
        ).wait()

    def transpose_block(s):
        g = gs[s]
        t = ts[s]
        for v in range(128):
            vc = jnp.full((16,), v, jnp.int32)
            for k in range(4):
                val = g[v, pl.ds(16 * k, 16)]
                plsc.store_scatter(t, [d_vecs[k], vc], val)

    def start_out(i, s):
        c = i // RBW
        b = RBW * wid + lax.rem(i, RBW)
        for a in range(8):
            pltpu.async_copy(
                ts[s].at[pl.ds(8 * a, 8), pl.ds(0, 128)],
                out_hbm.at[c, a, b],
                out_sem.at[s],
            )

    def wait_out(s):
        for a in range(8):
            pltpu.make_async_copy(
                ts[s].at[pl.ds(0, 8), pl.ds(0, 128)],
                out_hbm.at[0, a, 0],
                out_sem.at[s],
            ).wait()

    start_in(0, 0)
    start_in(1, 1)

    def outer(tt, carry):
        for s in range(2):
            i = 2 * tt + s
            wait_in(s)

            @pl.when(i >= 2)
            def _():
                wait_out(s)

            transpose_block(s)
            start_out(i, s)

            @pl.when(i + 2 < BLK_B)
            def _():
                start_in(i + 2, s)
        return carry

    lax.fori_loop(0, BLK_B // 2, outer, 0)
    wait_out(0)
    wait_out(1)


def kernel(x, W):
    wt = jnp.transpose(W)
    r = _tc_relayout(wt)
    table = r.reshape(V, DIM)
    idx = x.reshape(RB, 128, NUM_COLS).transpose(0, 2, 1).astype(jnp.int32)
    out5 = _gather_kernel(idx, table)
    return out5.transpose(2, 4, 0, 1, 3).reshape(NUM_ROWS, NUM_COLS, DIM)


# speedup vs baseline: 2.6146x; 1.2082x over previous
import functools

import jax
import jax.numpy as jnp
from jax import lax
from jax.experimental import pallas as pl
from jax.experimental.pallas import tpu as pltpu
from jax.experimental.pallas import tpu_sc as plsc

NUM_ROWS = 16384
NUM_COLS = 26
DIM = 64
V = 1000000

NC = 2
NS = 16
NW = NC * NS

RB = NUM_ROWS // 128
RBW = RB // NW
BLK_B = NUM_COLS * RBW

TCW = 2048
TC_GRID = (V + TCW - 1) // TCW
VPAD = TC_GRID * TCW


def _iota16():
    return lax.iota(jnp.int32, 16)


def _tc_relayout_body(wt_ref, idx_ref, out_ref, idxo_ref):
    w = wt_ref[...]
    out_ref[...] = jnp.concatenate(
        [w[:, : TCW // 2], w[:, TCW // 2 :]], axis=0
    ).T

    @pl.when(pl.program_id(0) == 0)
    def _():
        r = idx_ref[...]
        idxo_ref[...] = (r & -2048) | ((r & 1023) << 1) | ((r >> 10) & 1)


def _tc_relayout(wt, idx):
    return pl.pallas_call(
        _tc_relayout_body,
        grid=(TC_GRID,),
        in_specs=[
            pl.BlockSpec((DIM, TCW), lambda i: (0, i)),
            pl.BlockSpec((RB, NUM_COLS, 128), lambda i: (0, 0, 0)),
        ],
        out_specs=[
            pl.BlockSpec((TCW // 2, 128), lambda i: (i, 0)),
            pl.BlockSpec((RB, NUM_COLS, 128), lambda i: (0, 0, 0)),
        ],
        out_shape=[
            jax.ShapeDtypeStruct((VPAD // 2, 128), jnp.float32),
            jax.ShapeDtypeStruct((RB, NUM_COLS, 128), jnp.int32),
        ],
    )(wt, idx)


@functools.partial(
    pl.kernel,
    mesh=plsc.VectorSubcoreMesh(core_axis_name="c", subcore_axis_name="s"),
    out_type=jax.ShapeDtypeStruct((NUM_COLS, 8, RB, 8, 128), jnp.float32),
    scratch_types=[
        pltpu.VMEM((RBW, NUM_COLS, 128), jnp.int32),
        pltpu.VMEM((128, DIM), jnp.float32),
        pltpu.VMEM((128, DIM), jnp.float32),
        pltpu.VMEM((DIM, 129), jnp.float32),
        pltpu.VMEM((DIM, 129), jnp.float32),
        pltpu.SemaphoreType.DMA((2,)),
        pltpu.SemaphoreType.DMA((2,)),
    ],
    compiler_params=pltpu.CompilerParams(
        use_tc_tiling_on_sc=False, needs_layout_passes=False,
        disable_bounds_checks=True,
    ),
)
def _gather_kernel(idx_hbm, table_hbm, out_hbm, idx_v, g0, g1, t0, t1,
                   in_sem, out_sem):
    wid = lax.axis_index("s") * NC + lax.axis_index("c")
    gs = (g0, g1)
    ts = (t0, t1)
    iota = _iota16()
    d_vecs = [16 * k + iota for k in range(4)]

    pltpu.sync_copy(idx_hbm.at[pl.ds(RBW * wid, RBW)], idx_v)

    def start_in(i, s):
        c = i // RBW
        bl = lax.rem(i, RBW)
        pltpu.async_copy(
            table_hbm.at[idx_v.at[bl, c]], gs[s], in_sem.at[s]
        )

    def wait_in(s):
        pltpu.make_async_copy(
            table_hbm.at[idx_v.at[0, 0]], gs[s], in_sem.at[s]
        ).wait()

    def transpose_block(s):
        g = gs[s]
        t = ts[s]
        for v in range(128):
            vc = jnp.full((16,), v, jnp.int32)
            for k in range(4):
                val = g[v, pl.ds(16 * k, 16)]
                plsc.store_scatter(t, [d_vecs[k], vc], val)

    def start_out(i, s):
        c = i // RBW
        b = RBW * wid + lax.rem(i, RBW)
        for a in range(8):
            pltpu.async_copy(
                ts[s].at[pl.ds(8 * a, 8), pl.ds(0, 128)],
                out_hbm.at[c, a, b],
                out_sem.at[s],
            )

    def wait_out(s):
        for a in range(8):
            pltpu.make_async_copy(
                ts[s].at[pl.ds(0, 8), pl.ds(0, 128)],
                out_hbm.at[0, a, 0],
                out_sem.at[s],
            ).wait()

    start_in(0, 0)
    start_in(1, 1)

    def outer(tt, carry):
        for s in range(2):
            i = 2 * tt + s
            wait_in(s)

            @pl.when(i >= 2)
            def _():
                wait_out(s)

            transpose_block(s)
            start_out(i, s)

            @pl.when(i + 2 < BLK_B)
            def _():
                start_in(i + 2, s)
        return carry

    lax.fori_loop(0, BLK_B // 2, outer, 0)
    wait_out(0)
    wait_out(1)


def kernel(x, W):
    wt = jnp.transpose(W)
    idx0 = x.reshape(RB, 128, NUM_COLS).transpose(0, 2, 1).astype(jnp.int32)
    r, idx = _tc_relayout(wt, idx0)
    table = r.reshape(VPAD, DIM)
    out5 = _gather_kernel(idx, table)
    return out5.transpose(2, 4, 0, 1, 3).reshape(NUM_ROWS, NUM_COLS, DIM)


# speedup vs baseline: 3.1371x; 1.1998x over previous
import functools

import jax
import jax.numpy as jnp
from jax import lax
from jax.experimental import pallas as pl
from jax.experimental.pallas import tpu as pltpu
from jax.experimental.pallas import tpu_sc as plsc

NUM_ROWS = 16384
NUM_COLS = 26
DIM = 64
V = 1000000

NC = 2
NS = 16
NW = NC * NS

RB = NUM_ROWS // 128
RBW = RB // NW
BLK_B = NUM_COLS * RBW

TCW = 4096
TC_HALF = TCW // 2
TC_SHIFT = TC_HALF.bit_length() - 1
TC_GRID = (V + TCW - 1) // TCW
VPAD = TC_GRID * TCW


def _iota16():
    return lax.iota(jnp.int32, 16)


def _tc_relayout_body(wt_ref, idx_ref, out_ref, idxo_ref):
    w = wt_ref[...]
    out_ref[...] = jnp.concatenate(
        [w[:, :TC_HALF], w[:, TC_HALF:]], axis=0
    ).T

    @pl.when(pl.program_id(0) == 0)
    def _():
        r = idx_ref[...]
        idxo_ref[...] = (
            (r & -TCW) | ((r & (TC_HALF - 1)) << 1) | ((r >> TC_SHIFT) & 1)
        )


def _tc_relayout(wt, idx):
    return pl.pallas_call(
        _tc_relayout_body,
        grid=(TC_GRID,),
        in_specs=[
            pl.BlockSpec((DIM, TCW), lambda i: (0, i)),
            pl.BlockSpec((RB, NUM_COLS, 128), lambda i: (0, 0, 0)),
        ],
        out_specs=[
            pl.BlockSpec((TC_HALF, 128), lambda i: (i, 0)),
            pl.BlockSpec((RB, NUM_COLS, 128), lambda i: (0, 0, 0)),
        ],
        out_shape=[
            jax.ShapeDtypeStruct((VPAD // 2, 128), jnp.float32),
            jax.ShapeDtypeStruct((RB, NUM_COLS, 128), jnp.int32),
        ],
    )(wt, idx)


@functools.partial(
    pl.kernel,
    mesh=plsc.VectorSubcoreMesh(core_axis_name="c", subcore_axis_name="s"),
    out_type=jax.ShapeDtypeStruct((NUM_COLS, 8, RB, 8, 128), jnp.float32),
    scratch_types=[
        pltpu.VMEM((RBW, NUM_COLS, 128), jnp.int32),
        pltpu.VMEM((128, DIM), jnp.float32),
        pltpu.VMEM((128, DIM), jnp.float32),
        pltpu.VMEM((DIM, 129), jnp.float32),
        pltpu.VMEM((DIM, 129), jnp.float32),
        pltpu.SemaphoreType.DMA((2,)),
        pltpu.SemaphoreType.DMA((2,)),
    ],
    compiler_params=pltpu.CompilerParams(
        use_tc_tiling_on_sc=False, needs_layout_passes=False,
        disable_bounds_checks=True,
    ),
)
def _gather_kernel(idx_hbm, table_hbm, out_hbm, idx_v, g0, g1, t0, t1,
                   in_sem, out_sem):
    wid = lax.axis_index("s") * NC + lax.axis_index("c")
    gs = (g0, g1)
    ts = (t0, t1)
    iota = _iota16()
    d_vecs = [16 * k + iota for k in range(4)]

    pltpu.sync_copy(idx_hbm.at[pl.ds(RBW * wid, RBW)], idx_v)

    def start_in(i, s):
        c = i // RBW
        bl = lax.rem(i, RBW)
        pltpu.async_copy(
            table_hbm.at[idx_v.at[bl, c]], gs[s], in_sem.at[s]
        )

    def wait_in(s):
        pltpu.make_async_copy(
            table_hbm.at[idx_v.at[0, 0]], gs[s], in_sem.at[s]
        ).wait()

    def transpose_block(s):
        g = gs[s]
        t = ts[s]
        for v in range(128):
            vc = jnp.full((16,), v, jnp.int32)
            for k in range(4):
                val = g[v, pl.ds(16 * k, 16)]
                plsc.store_scatter(t, [d_vecs[k], vc], val)

    def start_out(i, s):
        c = i // RBW
        b = RBW * wid + lax.rem(i, RBW)
        for a in range(8):
            pltpu.async_copy(
                ts[s].at[pl.ds(8 * a, 8), pl.ds(0, 128)],
                out_hbm.at[c, a, b],
                out_sem.at[s],
            )

    def wait_out(s):
        for a in range(8):
            pltpu.make_async_copy(
                ts[s].at[pl.ds(0, 8), pl.ds(0, 128)],
                out_hbm.at[0, a, 0],
                out_sem.at[s],
            ).wait()

    start_in(0, 0)
    start_in(1, 1)

    def outer(tt, carry):
        for s in range(2):
            i = 2 * tt + s
            wait_in(s)

            @pl.when(i >= 2)
            def _():
                wait_out(s)

            transpose_block(s)
            start_out(i, s)

            @pl.when(i + 2 < BLK_B)
            def _():
                start_in(i + 2, s)
        return carry

    lax.fori_loop(0, BLK_B // 2, outer, 0)
    wait_out(0)
    wait_out(1)


def kernel(x, W):
    wt = jnp.transpose(W)
    idx0 = x.reshape(RB, 128, NUM_COLS).transpose(0, 2, 1).astype(jnp.int32)
    r, idx = _tc_relayout(wt, idx0)
    table = r.reshape(VPAD, DIM)
    out5 = _gather_kernel(idx, table)
    return out5.transpose(2, 4, 0, 1, 3).reshape(NUM_ROWS, NUM_COLS, DIM)


# speedup vs baseline: 3.5953x; 1.1461x over previous
import functools

import jax
import jax.numpy as jnp
from jax import lax
from jax.experimental import pallas as pl
from jax.experimental.pallas import tpu as pltpu
from jax.experimental.pallas import tpu_sc as plsc

NUM_ROWS = 16384
NUM_COLS = 26
DIM = 64
V = 1000000

NC = 2
NS = 16
NW = NC * NS

RB = NUM_ROWS // 128
RBW = RB // NW
BLK_B = NUM_COLS * RBW

TCW = 8192
TC_HALF = TCW // 2
TC_SHIFT = TC_HALF.bit_length() - 1
TC_GRID = (V + TCW - 1) // TCW
VPAD = TC_GRID * TCW


def _iota16():
    return lax.iota(jnp.int32, 16)


def _tc_relayout_body(wt_ref, idx_ref, out_ref, idxo_ref):
    w = wt_ref[...]
    out_ref[...] = jnp.concatenate(
        [w[:, :TC_HALF], w[:, TC_HALF:]], axis=0
    ).T

    @pl.when(pl.program_id(0) == 0)
    def _():
        r = idx_ref[...]
        idxo_ref[...] = (
            (r & -TCW) | ((r & (TC_HALF - 1)) << 1) | ((r >> TC_SHIFT) & 1)
        )


def _tc_relayout(wt, idx):
    return pl.pallas_call(
        _tc_relayout_body,
        grid=(TC_GRID,),
        in_specs=[
            pl.BlockSpec((DIM, TCW), lambda i: (0, i)),
            pl.BlockSpec((RB, NUM_COLS, 128), lambda i: (0, 0, 0)),
        ],
        out_specs=[
            pl.BlockSpec((TC_HALF, 128), lambda i: (i, 0)),
            pl.BlockSpec((RB, NUM_COLS, 128), lambda i: (0, 0, 0)),
        ],
        out_shape=[
            jax.ShapeDtypeStruct((VPAD // 2, 128), jnp.float32),
            jax.ShapeDtypeStruct((RB, NUM_COLS, 128), jnp.int32),
        ],
    )(wt, idx)


@functools.partial(
    pl.kernel,
    mesh=plsc.VectorSubcoreMesh(core_axis_name="c", subcore_axis_name="s"),
    out_type=jax.ShapeDtypeStruct((NUM_COLS, 8, RB, 8, 128), jnp.float32),
    scratch_types=[
        pltpu.VMEM((RBW, NUM_COLS, 128), jnp.int32),
        pltpu.VMEM((128, DIM), jnp.float32),
        pltpu.VMEM((128, DIM), jnp.float32),
        pltpu.VMEM((DIM, 129), jnp.float32),
        pltpu.VMEM((DIM, 129), jnp.float32),
        pltpu.SemaphoreType.DMA((2,)),
        pltpu.SemaphoreType.DMA((2,)),
    ],
    compiler_params=pltpu.CompilerParams(
        use_tc_tiling_on_sc=False, needs_layout_passes=False,
        disable_bounds_checks=True,
    ),
)
def _gather_kernel(idx_hbm, table_hbm, out_hbm, idx_v, g0, g1, t0, t1,
                   in_sem, out_sem):
    wid = lax.axis_index("s") * NC + lax.axis_index("c")
    gs = (g0, g1)
    ts = (t0, t1)
    iota = _iota16()
    d_vecs = [16 * k + iota for k in range(4)]

    pltpu.sync_copy(idx_hbm.at[pl.ds(RBW * wid, RBW)], idx_v)

    def start_in(i, s):
        c = i // RBW
        bl = lax.rem(i, RBW)
        pltpu.async_copy(
            table_hbm.at[idx_v.at[bl, c]], gs[s], in_sem.at[s]
        )

    def wait_in(s):
        pltpu.make_async_copy(
            table_hbm.at[idx_v.at[0, 0]], gs[s], in_sem.at[s]
        ).wait()

    def transpose_block(s):
        g = gs[s]
        t = ts[s]
        for v in range(128):
            vc = jnp.full((16,), v, jnp.int32)
            for k in range(4):
                val = g[v, pl.ds(16 * k, 16)]
                plsc.store_scatter(t, [d_vecs[k], vc], val)

    def start_out(i, s):
        c = i // RBW
        b = RBW * wid + lax.rem(i, RBW)
        for a in range(8):
            pltpu.async_copy(
                ts[s].at[pl.ds(8 * a, 8), pl.ds(0, 128)],
                out_hbm.at[c, a, b],
                out_sem.at[s],
            )

    def wait_out(s):
        for a in range(8):
            pltpu.make_async_copy(
                ts[s].at[pl.ds(0, 8), pl.ds(0, 128)],
                out_hbm.at[0, a, 0],
                out_sem.at[s],
            ).wait()

    start_in(0, 0)
    start_in(1, 1)

    def outer(tt, carry):
        for s in range(2):
            i = 2 * tt + s
            wait_in(s)

            @pl.when(i >= 2)
            def _():
                wait_out(s)

            transpose_block(s)
            start_out(i, s)

            @pl.when(i + 2 < BLK_B)
            def _():
                start_in(i + 2, s)
        return carry

    lax.fori_loop(0, BLK_B // 2, outer, 0)
    wait_out(0)
    wait_out(1)


def kernel(x, W):
    wt = jnp.transpose(W)
    idx0 = x.reshape(RB, 128, NUM_COLS).transpose(0, 2, 1).astype(jnp.int32)
    r, idx = _tc_relayout(wt, idx0)
    table = r.reshape(VPAD, DIM)
    out5 = _gather_kernel(idx, table)
    return out5.transpose(2, 4, 0, 1, 3).reshape(NUM_ROWS, NUM_COLS, DIM)


# speedup vs baseline: 3.8206x; 1.0626x over previous
import functools

import jax
import jax.numpy as jnp
from jax import lax
from jax.experimental import pallas as pl
from jax.experimental.pallas import tpu as pltpu
from jax.experimental.pallas import tpu_sc as plsc

NUM_ROWS = 16384
NUM_COLS = 26
DIM = 64
V = 1000000

NC = 2
NS = 16
NW = NC * NS

RB = NUM_ROWS // 128
RBW = RB // NW
BLK_B = NUM_COLS * RBW

TCW = 16384
TC_HALF = TCW // 2
TC_SHIFT = TC_HALF.bit_length() - 1
TC_GRID = (V + TCW - 1) // TCW
VPAD = TC_GRID * TCW


def _iota16():
    return lax.iota(jnp.int32, 16)


def _tc_relayout_body(wt_ref, idx_ref, out_ref, idxo_ref):
    w = wt_ref[...]
    out_ref[...] = jnp.concatenate(
        [w[:, :TC_HALF], w[:, TC_HALF:]], axis=0
    ).T

    @pl.when(pl.program_id(0) == 0)
    def _():
        r = idx_ref[...]
        idxo_ref[...] = (
            (r & -TCW) | ((r & (TC_HALF - 1)) << 1) | ((r >> TC_SHIFT) & 1)
        )


def _tc_relayout(wt, idx):
    return pl.pallas_call(
        _tc_relayout_body,
        grid=(TC_GRID,),
        in_specs=[
            pl.BlockSpec((DIM, TCW), lambda i: (0, i)),
            pl.BlockSpec((RB, NUM_COLS, 128), lambda i: (0, 0, 0)),
        ],
        out_specs=[
            pl.BlockSpec((TC_HALF, 128), lambda i: (i, 0)),
            pl.BlockSpec((RB, NUM_COLS, 128), lambda i: (0, 0, 0)),
        ],
        out_shape=[
            jax.ShapeDtypeStruct((VPAD // 2, 128), jnp.float32),
            jax.ShapeDtypeStruct((RB, NUM_COLS, 128), jnp.int32),
        ],
    )(wt, idx)


@functools.partial(
    pl.kernel,
    mesh=plsc.VectorSubcoreMesh(core_axis_name="c", subcore_axis_name="s"),
    out_type=jax.ShapeDtypeStruct((NUM_COLS, 8, RB, 8, 128), jnp.float32),
    scratch_types=[
        pltpu.VMEM((RBW, NUM_COLS, 128), jnp.int32),
        pltpu.VMEM((128, DIM), jnp.float32),
        pltpu.VMEM((128, DIM), jnp.float32),
        pltpu.VMEM((DIM, 129), jnp.float32),
        pltpu.VMEM((DIM, 129), jnp.float32),
        pltpu.SemaphoreType.DMA((2,)),
        pltpu.SemaphoreType.DMA((2,)),
    ],
    compiler_params=pltpu.CompilerParams(
        use_tc_tiling_on_sc=False, needs_layout_passes=False,
        disable_bounds_checks=True,
    ),
)
def _gather_kernel(idx_hbm, table_hbm, out_hbm, idx_v, g0, g1, t0, t1,
                   in_sem, out_sem):
    wid = lax.axis_index("s") * NC + lax.axis_index("c")
    gs = (g0, g1)
    ts = (t0, t1)
    iota = _iota16()
    d_vecs = [16 * k + iota for k in range(4)]

    pltpu.sync_copy(idx_hbm.at[pl.ds(RBW * wid, RBW)], idx_v)

    def start_in(i, s):
        c = i // RBW
        bl = lax.rem(i, RBW)
        pltpu.async_copy(
            table_hbm.at[idx_v.at[bl, c]], gs[s], in_sem.at[s]
        )

    def wait_in(s):
        pltpu.make_async_copy(
            table_hbm.at[idx_v.at[0, 0]], gs[s], in_sem.at[s]
        ).wait()

    def transpose_block(s):
        g = gs[s]
        t = ts[s]
        for v in range(128):
            vc = jnp.full((16,), v, jnp.int32)
            for k in range(4):
                val = g[v, pl.ds(16 * k, 16)]
                plsc.store_scatter(t, [d_vecs[k], vc], val)

    def start_out(i, s):
        c = i // RBW
        b = RBW * wid + lax.rem(i, RBW)
        for a in range(8):
            pltpu.async_copy(
                ts[s].at[pl.ds(8 * a, 8), pl.ds(0, 128)],
                out_hbm.at[c, a, b],
                out_sem.at[s],
            )

    def wait_out(s):
        for a in range(8):
            pltpu.make_async_copy(
                ts[s].at[pl.ds(0, 8), pl.ds(0, 128)],
                out_hbm.at[0, a, 0],
                out_sem.at[s],
            ).wait()

    start_in(0, 0)
    start_in(1, 1)

    def outer(tt, carry):
        for s in range(2):
            i = 2 * tt + s
            wait_in(s)

            @pl.when(i >= 2)
            def _():
                wait_out(s)

            transpose_block(s)
            start_out(i, s)

            @pl.when(i + 2 < BLK_B)
            def _():
                start_in(i + 2, s)
        return carry

    lax.fori_loop(0, BLK_B // 2, outer, 0)
    wait_out(0)
    wait_out(1)


def kernel(x, W):
    wt = jnp.transpose(W)
    idx0 = x.reshape(RB, 128, NUM_COLS).transpose(0, 2, 1).astype(jnp.int32)
    r, idx = _tc_relayout(wt, idx0)
    table = r.reshape(VPAD, DIM)
    out5 = _gather_kernel(idx, table)
    return out5.transpose(2, 4, 0, 1, 3).reshape(NUM_ROWS, NUM_COLS, DIM)
